# baseline (speedup 1.0000x reference)
import functools

import jax
import jax.numpy as jnp
from jax import lax
from jax.experimental import pallas as pl
from jax.experimental.pallas import tpu as pltpu
from jax.experimental.pallas import tpu_sc as plsc

BATCH = 16384
EMB = 32
NC = 2
NS = 16
NW = NC * NS
BPW = BATCH // NW
GRP = 8
CHUNK = 128
LANES = 16


def _extract(slabs, g, iv, ext, k, sem):
  for r in range(GRP):
    col = jnp.broadcast_to(iv[r] & 127, (LANES,))
    rows0 = lax.iota(jnp.int32, LANES)
    sl = slabs.at[g * GRP + r]
    lo = plsc.load_gather(sl, [rows0, col])
    hi = plsc.load_gather(sl, [rows0 + LANES, col])
    ext[k + r, pl.ds(0, LANES)] = lo
    ext[k + r, pl.ds(LANES, LANES)] = hi


def _sc_gather_body(user_h, item_h, ug_h, ig_h, um_h, im_h,
                    oug_h, oig_h, oum_h, oim_h,
                    uvmem, ivmem, slabs, ext0, ext1, sem, osem0, osem1):
  wid = lax.axis_index("s") * NC + lax.axis_index("c")
  base = wid * BPW
  pltpu.sync_copy(user_h.at[pl.ds(base, BPW)], uvmem)
  pltpu.sync_copy(item_h.at[pl.ds(base, BPW)], ivmem)

  del ext1, osem1
  ext = ext0
  sems = (sem, osem0)
  n_grp = BPW // (2 * GRP)
  grp_per_chunk = CHUNK // (2 * GRP)

  for tbl, idxs, out_h in ((ug_h, uvmem, oug_h), (ig_h, ivmem, oig_h),
                           (um_h, uvmem, oum_h), (im_h, ivmem, oim_h)):

    def fire(g, iv, tbl=tbl):
      for r in range(GRP):
        src_col = pl.multiple_of(iv[r] & ~jnp.int32(127), 128)
        pltpu.async_copy(tbl.at[:, pl.ds(src_col, 128)],
                         slabs.at[g * GRP + r], sems[g])

    def drain(g, tbl=tbl):
      pltpu.make_async_copy(tbl.at[:, pl.ds(0, GRP * 128)],
                            slabs.at[pl.ds(g * GRP, GRP)], sems[g]).wait()

    iv_p = idxs[pl.ds(0, 2 * GRP)]
    fire(0, [iv_p[r] for r in range(GRP)])
    fire(1, [iv_p[r + GRP] for r in range(GRP)])

    def body(c, iv_pend, idxs=idxs, tbl=tbl, out_h=out_h):
      off = jnp.minimum(c * (2 * GRP), BPW - 2 * GRP)
      iv_new = idxs[pl.ds(off, 2 * GRP)]
      k0 = (c - 1) * (2 * GRP)
      krow = lax.rem(k0, CHUNK)
      drain(0)
      _extract(slabs, 0, [iv_pend[r] for r in range(GRP)], ext, krow, sem)

      @pl.when(c < n_grp)
      def _():
        fire(0, [iv_new[r] for r in range(GRP)])

      drain(1)
      _extract(slabs, 1, [iv_pend[r + GRP] for r in range(GRP)], ext,
               krow + GRP, sem)

      @pl.when(c < n_grp)
      def _():
        fire(1, [iv_new[r + GRP] for r in range(GRP)])

      @pl.when(lax.rem(c, grp_per_chunk) == 0)
      def _():
        chunk = c // grp_per_chunk - 1
        pltpu.sync_copy(ext,
                        out_h.at[pl.ds(base + chunk * CHUNK, CHUNK)])

      return iv_new

    lax.fori_loop(1, n_grp + 1, body, iv_p, unroll=False)


@functools.lru_cache(maxsize=None)
def _sc_gather():
  return functools.partial(
      pl.kernel,
      out_type=(jax.ShapeDtypeStruct((BATCH, EMB), jnp.float32),) * 4,
      mesh=plsc.VectorSubcoreMesh(core_axis_name="c", subcore_axis_name="s",
                                  num_cores=NC, num_subcores=NS),
      compiler_params=pltpu.CompilerParams(use_tc_tiling_on_sc=True,
                                           needs_layout_passes=False),
      scratch_types=[
          pltpu.VMEM((BPW,), jnp.int32),
          pltpu.VMEM((BPW,), jnp.int32),
          pltpu.VMEM((2 * GRP, EMB, 128), jnp.float32),
          pltpu.VMEM((CHUNK, EMB), jnp.float32),
          pltpu.VMEM((CHUNK, EMB), jnp.float32),
          pltpu.SemaphoreType.DMA,
          pltpu.SemaphoreType.DMA,
          pltpu.SemaphoreType.DMA,
      ],
  )(_sc_gather_body)


BLK = 2048


def _mlp_body(ug_ref, ig_ref, mu_ref, mi_ref,
              w1a_ref, w1b_ref, w2_ref, w3_ref, woa_ref, wob_ref,
              b1_ref, b2_ref, b3_ref, bo_ref, out_ref):
  mu = mu_ref[...]
  mi = mi_ref[...]
  h1 = jnp.dot(mu, w1a_ref[...], preferred_element_type=jnp.float32)
  h1 = h1 + jnp.dot(mi, w1b_ref[...], preferred_element_type=jnp.float32)
  h1 = jnp.maximum(h1 + b1_ref[...], 0.0)
  h2 = jnp.maximum(
      jnp.dot(h1, w2_ref[...], preferred_element_type=jnp.float32)
      + b2_ref[...], 0.0)
  h3 = jnp.maximum(
      jnp.dot(h2, w3_ref[...], preferred_element_type=jnp.float32)
      + b3_ref[...], 0.0)
  gmf = ug_ref[...] * ig_ref[...]
  logit = (jnp.dot(gmf, woa_ref[...], preferred_element_type=jnp.float32)
           + jnp.dot(h3, wob_ref[...], preferred_element_type=jnp.float32)
           + bo_ref[...])
  out_ref[...] = 1.0 / (1.0 + jnp.exp(-logit))


def _mlp_call(ug, ig, mu, mi, w1a, w1b, w2t, w3t, woa, wob, b1, b2, b3, bo):
  grid = (BATCH // BLK,)
  bspec = pl.BlockSpec((BLK, EMB), lambda i: (i, 0))
  wspec = lambda shape: pl.BlockSpec(shape, lambda i: (0, 0))
  return pl.pallas_call(
      _mlp_body,
      grid=grid,
      in_specs=[bspec, bspec, bspec, bspec,
                wspec((EMB, 64)), wspec((EMB, 64)), wspec((64, 32)),
                wspec((32, 16)), wspec((EMB, 1)), wspec((16, 1)),
                wspec((1, 64)), wspec((1, 32)), wspec((1, 16)),
                wspec((1, 1))],
      out_specs=pl.BlockSpec((BLK, 1), lambda i: (i, 0)),
      out_shape=jax.ShapeDtypeStruct((BATCH, 1), jnp.float32),
  )(ug, ig, mu, mi, w1a, w1b, w2t, w3t, woa, wob, b1, b2, b3, bo)


def kernel(user, item, Ug, Ig, Um, Im, W1, b1, W2, b2, W3, b3, Wo, bo):
  user = user.astype(jnp.int32)
  item = item.astype(jnp.int32)
  ug, ig, mu, mi = _sc_gather()(user, item, Ug.T, Ig.T, Um.T, Im.T)
  w1a = W1[:, :EMB].T
  w1b = W1[:, EMB:].T
  w2t = W2.T
  w3t = W3.T
  woa = Wo[:, :EMB].T
  wob = Wo[:, EMB:].T
  out = _mlp_call(ug, ig, mu, mi, w1a, w1b, w2t, w3t, woa, wob,
                  b1.reshape(1, 64), b2.reshape(1, 32), b3.reshape(1, 16),
                  bo.reshape(1, 1))
  return jnp.squeeze(out)


# speedup vs baseline: 1.0654x; 1.0654x over previous
import functools

import jax
import jax.numpy as jnp
from jax import lax
from jax.experimental import pallas as pl
from jax.experimental.pallas import tpu as pltpu
from jax.experimental.pallas import tpu_sc as plsc

BATCH = 16384
EMB = 32
NC = 2
NS = 16
NW = NC * NS
BPW = BATCH // NW
GRPL = 8
NSETS = 3
CHUNK = 128
LANES = 16


def _extract(slabs, g, iv, ext, k, sem):
  for r in range(GRPL):
    col = jnp.broadcast_to(iv[r] & 127, (LANES,))
    rows0 = lax.iota(jnp.int32, LANES)
    sl = slabs.at[g * GRPL + r]
    lo = plsc.load_gather(sl, [rows0, col])
    hi = plsc.load_gather(sl, [rows0 + LANES, col])
    ext[k + r, pl.ds(0, LANES)] = lo
    ext[k + r, pl.ds(LANES, LANES)] = hi


def _sc_gather_body(user_h, item_h, ug_h, ig_h, um_h, im_h,
                    oug_h, oig_h, oum_h, oim_h,
                    uvmem, ivmem, slabs, ext0, ext1, sem, osem0, osem1):
  wid = lax.axis_index("s") * NC + lax.axis_index("c")
  base = wid * BPW
  pltpu.sync_copy(user_h.at[pl.ds(base, BPW)], uvmem)
  pltpu.sync_copy(item_h.at[pl.ds(base, BPW)], ivmem)

  ext = ext0
  del ext1
  sems = (sem, osem0, osem1)
  n_grp = BPW // GRPL
  grp_per_chunk = CHUNK // GRPL

  for tbl, idxs, out_h in ((ug_h, uvmem, oug_h), (ig_h, ivmem, oig_h),
                           (um_h, uvmem, oum_h), (im_h, ivmem, oim_h)):

    def fire(ph, iv, tbl=tbl):
      for r in range(GRPL):
        src_col = pl.multiple_of(iv[r] & ~jnp.int32(127), 128)
        pltpu.async_copy(tbl.at[:, pl.ds(src_col, 128)],
                         slabs.at[ph * GRPL + r], sems[ph])

    def drain(ph, tbl=tbl):
      pltpu.make_async_copy(tbl.at[:, pl.ds(0, GRPL * 128)],
                            slabs.at[pl.ds(ph * GRPL, GRPL)],
                            sems[ph]).wait()

    def load_iv(g, idxs=idxs):
      off = jnp.minimum(g * GRPL, BPW - LANES)
      return idxs[pl.ds(off, LANES)]

    carry = []
    for ph in range(NSETS):
      iv0 = load_iv(ph)
      fire(ph, [iv0[r] for r in range(GRPL)])
      carry.append(iv0)

    def body(c, iv_pend, tbl=tbl, out_h=out_h):
      new_carry = []
      for ph in range(NSETS):
        e = NSETS * (c - 1) + ph
        f = NSETS * c + ph
        iv_e = iv_pend[ph]

        @pl.when(e < n_grp)
        def _(e=e, iv_e=iv_e):
          drain(ph)
          krow = lax.rem(e * GRPL, CHUNK)
          _extract(slabs, ph, [iv_e[r] for r in range(GRPL)], ext, krow,
                   sem)

        iv_f = load_iv(f)

        @pl.when(f < n_grp)
        def _(iv_f=iv_f):
          fire(ph, [iv_f[r] for r in range(GRPL)])

        @pl.when(jnp.logical_and(e < n_grp,
                                 lax.rem(e + 1, grp_per_chunk) == 0))
        def _(e=e):
          chunk = (e + 1) // grp_per_chunk - 1
          pltpu.sync_copy(ext,
                          out_h.at[pl.ds(base + chunk * CHUNK, CHUNK)])

        new_carry.append(iv_f)
      return tuple(new_carry)

    n_iter = (n_grp + NSETS - 1) // NSETS + 1
    lax.fori_loop(1, n_iter + 1, body, tuple(carry), unroll=False)


@functools.lru_cache(maxsize=None)
def _sc_gather():
  return functools.partial(
      pl.kernel,
      out_type=(jax.ShapeDtypeStruct((BATCH, EMB), jnp.float32),) * 4,
      mesh=plsc.VectorSubcoreMesh(core_axis_name="c", subcore_axis_name="s",
                                  num_cores=NC, num_subcores=NS),
      compiler_params=pltpu.CompilerParams(use_tc_tiling_on_sc=True,
                                           needs_layout_passes=False),
      scratch_types=[
          pltpu.VMEM((BPW,), jnp.int32),
          pltpu.VMEM((BPW,), jnp.int32),
          pltpu.VMEM((NSETS * GRPL, EMB, 128), jnp.float32),
          pltpu.VMEM((CHUNK, EMB), jnp.float32),
          pltpu.VMEM((CHUNK, EMB), jnp.float32),
          pltpu.SemaphoreType.DMA,
          pltpu.SemaphoreType.DMA,
          pltpu.SemaphoreType.DMA,
      ],
  )(_sc_gather_body)


BLK = 2048


def _mlp_body(ug_ref, ig_ref, mu_ref, mi_ref,
              w1a_ref, w1b_ref, w2_ref, w3_ref, woa_ref, wob_ref,
              b1_ref, b2_ref, b3_ref, bo_ref, out_ref):
  mu = mu_ref[...]
  mi = mi_ref[...]
  h1 = jnp.dot(mu, w1a_ref[...], preferred_element_type=jnp.float32)
  h1 = h1 + jnp.dot(mi, w1b_ref[...], preferred_element_type=jnp.float32)
  h1 = jnp.maximum(h1 + b1_ref[...], 0.0)
  h2 = jnp.maximum(
      jnp.dot(h1, w2_ref[...], preferred_element_type=jnp.float32)
      + b2_ref[...], 0.0)
  h3 = jnp.maximum(
      jnp.dot(h2, w3_ref[...], preferred_element_type=jnp.float32)
      + b3_ref[...], 0.0)
  gmf = ug_ref[...] * ig_ref[...]
  logit = (jnp.dot(gmf, woa_ref[...], preferred_element_type=jnp.float32)
           + jnp.dot(h3, wob_ref[...], preferred_element_type=jnp.float32)
           + bo_ref[...])
  out_ref[...] = 1.0 / (1.0 + jnp.exp(-logit))


def _mlp_call(ug, ig, mu, mi, w1a, w1b, w2t, w3t, woa, wob, b1, b2, b3, bo):
  grid = (BATCH // BLK,)
  bspec = pl.BlockSpec((BLK, EMB), lambda i: (i, 0))
  wspec = lambda shape: pl.BlockSpec(shape, lambda i: (0, 0))
  return pl.pallas_call(
      _mlp_body,
      grid=grid,
      in_specs=[bspec, bspec, bspec, bspec,
                wspec((EMB, 64)), wspec((EMB, 64)), wspec((64, 32)),
                wspec((32, 16)), wspec((EMB, 1)), wspec((16, 1)),
                wspec((1, 64)), wspec((1, 32)), wspec((1, 16)),
                wspec((1, 1))],
      out_specs=pl.BlockSpec((BLK, 1), lambda i: (i, 0)),
      out_shape=jax.ShapeDtypeStruct((BATCH, 1), jnp.float32),
  )(ug, ig, mu, mi, w1a, w1b, w2t, w3t, woa, wob, b1, b2, b3, bo)


def kernel(user, item, Ug, Ig, Um, Im, W1, b1, W2, b2, W3, b3, Wo, bo):
  user = user.astype(jnp.int32)
  item = item.astype(jnp.int32)
  ug, ig, mu, mi = _sc_gather()(user, item, Ug.T, Ig.T, Um.T, Im.T)
  w1a = W1[:, :EMB].T
  w1b = W1[:, EMB:].T
  w2t = W2.T
  w3t = W3.T
  woa = Wo[:, :EMB].T
  wob = Wo[:, EMB:].T
  out = _mlp_call(ug, ig, mu, mi, w1a, w1b, w2t, w3t, woa, wob,
                  b1.reshape(1, 64), b2.reshape(1, 32), b3.reshape(1, 16),
                  bo.reshape(1, 1))
  return jnp.squeeze(out)


# speedup vs baseline: 1.0677x; 1.0022x over previous
import functools

import jax
import jax.numpy as jnp
from jax import lax
from jax.experimental import pallas as pl
from jax.experimental.pallas import tpu as pltpu
from jax.experimental.pallas import tpu_sc as plsc

BATCH = 16384
EMB = 32
NC = 2
NS = 16
NW = NC * NS
BPW = BATCH // NW
GRPL = 8
NSETS = 3
CHUNK = 128
LANES = 16


def _extract(slabs, g, iv, ext, k, sem):
  for r in range(GRPL):
    col = jnp.broadcast_to(iv[r] & 127, (LANES,))
    rows0 = lax.iota(jnp.int32, LANES)
    sl = slabs.at[g * GRPL + r]
    lo = plsc.load_gather(sl, [rows0, col])
    hi = plsc.load_gather(sl, [rows0 + LANES, col])
    ext[k + r, pl.ds(0, LANES)] = lo
    ext[k + r, pl.ds(LANES, LANES)] = hi


def _sc_gather_body(user_h, item_h, ug_h, ig_h, um_h, im_h,
                    oug_h, oig_h, oum_h, oim_h,
                    uvmem, ivmem, slabs, ext0, ext1, sem, osem0, osem1):
  wid = lax.axis_index("s") * NC + lax.axis_index("c")
  base = wid * BPW
  pltpu.sync_copy(user_h.at[pl.ds(base, BPW)], uvmem)
  pltpu.sync_copy(item_h.at[pl.ds(base, BPW)], ivmem)

  ext = ext0
  del ext1
  sems = (sem, osem0, osem1)
  n_grp = BPW // GRPL
  grp_per_chunk = CHUNK // GRPL

  for tbl, idxs, out_h in ((ug_h, uvmem, oug_h), (ig_h, ivmem, oig_h),
                           (um_h, uvmem, oum_h), (im_h, ivmem, oim_h)):

    def fire(ph, iv, tbl=tbl):
      for r in range(GRPL):
        src_col = pl.multiple_of(iv[r] & ~jnp.int32(127), 128)
        pltpu.async_copy(tbl.at[:, pl.ds(src_col, 128)],
                         slabs.at[ph * GRPL + r], sems[ph])

    def drain(ph, tbl=tbl):
      pltpu.make_async_copy(tbl.at[:, pl.ds(0, GRPL * 128)],
                            slabs.at[pl.ds(ph * GRPL, GRPL)],
                            sems[ph]).wait()

    def load_iv(g, idxs=idxs):
      off = jnp.minimum(g * GRPL, BPW - LANES)
      return idxs[pl.ds(off, LANES)]

    carry = []
    for ph in range(NSETS):
      iv0 = load_iv(ph)
      fire(ph, [iv0[r] for r in range(GRPL)])
      carry.append(iv0)

    def body(c, iv_pend, tbl=tbl, out_h=out_h):
      new_carry = []
      def scalars(g, iv):
        hi = g * GRPL > BPW - LANES
        return [jnp.where(hi, iv[r + LANES - GRPL], iv[r])
                for r in range(GRPL)]

      for ph in range(NSETS):
        e = NSETS * (c - 1) + ph
        f = NSETS * c + ph
        iv_e = iv_pend[ph]

        @pl.when(e < n_grp)
        def _(e=e, iv_e=iv_e):
          drain(ph)
          krow = lax.rem(e * GRPL, CHUNK)
          _extract(slabs, ph, scalars(e, iv_e), ext, krow, sem)

        iv_f = load_iv(f)

        @pl.when(f < n_grp)
        def _(f=f, iv_f=iv_f):
          fire(ph, scalars(f, iv_f))

        @pl.when(jnp.logical_and(e < n_grp,
                                 lax.rem(e + 1, grp_per_chunk) == 0))
        def _(e=e):
          chunk = (e + 1) // grp_per_chunk - 1
          pltpu.sync_copy(ext,
                          out_h.at[pl.ds(base + chunk * CHUNK, CHUNK)])

        new_carry.append(iv_f)
      return tuple(new_carry)

    n_iter = (n_grp + NSETS - 1) // NSETS + 1
    lax.fori_loop(1, n_iter + 1, body, tuple(carry), unroll=False)


@functools.lru_cache(maxsize=None)
def _sc_gather():
  return functools.partial(
      pl.kernel,
      out_type=(jax.ShapeDtypeStruct((BATCH, EMB), jnp.float32),) * 4,
      mesh=plsc.VectorSubcoreMesh(core_axis_name="c", subcore_axis_name="s",
                                  num_cores=NC, num_subcores=NS),
      compiler_params=pltpu.CompilerParams(use_tc_tiling_on_sc=True,
                                           needs_layout_passes=False),
      scratch_types=[
          pltpu.VMEM((BPW,), jnp.int32),
          pltpu.VMEM((BPW,), jnp.int32),
          pltpu.VMEM((NSETS * GRPL, EMB, 128), jnp.float32),
          pltpu.VMEM((CHUNK, EMB), jnp.float32),
          pltpu.VMEM((CHUNK, EMB), jnp.float32),
          pltpu.SemaphoreType.DMA,
          pltpu.SemaphoreType.DMA,
          pltpu.SemaphoreType.DMA,
      ],
  )(_sc_gather_body)


BLK = 2048


def _mlp_body(ug_ref, ig_ref, mu_ref, mi_ref,
              w1a_ref, w1b_ref, w2_ref, w3_ref, woa_ref, wob_ref,
              b1_ref, b2_ref, b3_ref, bo_ref, out_ref):
  mu = mu_ref[...]
  mi = mi_ref[...]
  h1 = jnp.dot(mu, w1a_ref[...], preferred_element_type=jnp.float32)
  h1 = h1 + jnp.dot(mi, w1b_ref[...], preferred_element_type=jnp.float32)
  h1 = jnp.maximum(h1 + b1_ref[...], 0.0)
  h2 = jnp.maximum(
      jnp.dot(h1, w2_ref[...], preferred_element_type=jnp.float32)
      + b2_ref[...], 0.0)
  h3 = jnp.maximum(
      jnp.dot(h2, w3_ref[...], preferred_element_type=jnp.float32)
      + b3_ref[...], 0.0)
  gmf = ug_ref[...] * ig_ref[...]
  logit = (jnp.dot(gmf, woa_ref[...], preferred_element_type=jnp.float32)
           + jnp.dot(h3, wob_ref[...], preferred_element_type=jnp.float32)
           + bo_ref[...])
  out_ref[...] = 1.0 / (1.0 + jnp.exp(-logit))


def _mlp_call(ug, ig, mu, mi, w1a, w1b, w2t, w3t, woa, wob, b1, b2, b3, bo):
  grid = (BATCH // BLK,)
  bspec = pl.BlockSpec((BLK, EMB), lambda i: (i, 0))
  wspec = lambda shape: pl.BlockSpec(shape, lambda i: (0, 0))
  return pl.pallas_call(
      _mlp_body,
      grid=grid,
      in_specs=[bspec, bspec, bspec, bspec,
                wspec((EMB, 64)), wspec((EMB, 64)), wspec((64, 32)),
                wspec((32, 16)), wspec((EMB, 1)), wspec((16, 1)),
                wspec((1, 64)), wspec((1, 32)), wspec((1, 16)),
                wspec((1, 1))],
      out_specs=pl.BlockSpec((BLK, 1), lambda i: (i, 0)),
      out_shape=jax.ShapeDtypeStruct((BATCH, 1), jnp.float32),
  )(ug, ig, mu, mi, w1a, w1b, w2t, w3t, woa, wob, b1, b2, b3, bo)


def kernel(user, item, Ug, Ig, Um, Im, W1, b1, W2, b2, W3, b3, Wo, bo):
  user = user.astype(jnp.int32)
  item = item.astype(jnp.int32)
  ug, ig, mu, mi = _sc_gather()(user, item, Ug.T, Ig.T, Um.T, Im.T)
  w1a = W1[:, :EMB].T
  w1b = W1[:, EMB:].T
  w2t = W2.T
  w3t = W3.T
  woa = Wo[:, :EMB].T
  wob = Wo[:, EMB:].T
  out = _mlp_call(ug, ig, mu, mi, w1a, w1b, w2t, w3t, woa, wob,
                  b1.reshape(1, 64), b2.reshape(1, 32), b3.reshape(1, 16),
                  bo.reshape(1, 1))
  return jnp.squeeze(out)
